# baseline (speedup 1.0000x reference)
import functools

import jax
import jax.numpy as jnp
from jax import lax
from jax.experimental import pallas as pl
from jax.experimental.pallas import tpu as pltpu
from jax.experimental.pallas import tpu_sc as plsc

D_MODEL = 64
LANE = 128
NUM_CORES = 2
NUM_SUBCORES = 16
NUM_WORKERS = NUM_CORES * NUM_SUBCORES


@functools.cache
def _build(seq, batch, vocab):
    assert batch == LANE * NUM_WORKERS and seq % 2 == 0
    n_bt = batch // LANE

    mesh = plsc.VectorSubcoreMesh(core_axis_name="c", subcore_axis_name="s")

    @functools.partial(
        pl.kernel,
        mesh=mesh,
        compiler_params=pltpu.CompilerParams(needs_layout_passes=False),
        out_type=jax.ShapeDtypeStruct((seq, 8, n_bt, 8, LANE), jnp.float32),
        scratch_types=[
            pltpu.VMEM((seq, LANE), jnp.int32),
            pltpu.VMEM((seq, LANE), jnp.int32),
            pltpu.VMEM((LANE, LANE), jnp.float32),
            pltpu.VMEM((LANE, LANE), jnp.float32),
            pltpu.VMEM((D_MODEL, LANE), jnp.float32),
            pltpu.VMEM((D_MODEL, LANE), jnp.float32),
            pltpu.SemaphoreType.DMA,
            pltpu.SemaphoreType.DMA,
            pltpu.SemaphoreType.DMA,
            pltpu.SemaphoreType.DMA,
        ],
    )
    def gather_kernel(tokp_hbm, tokh_hbm, table_hbm, out_hbm,
                      idxall, hall, g0, g1, ob0, ob1, sg0, sg1, so0, so1):
        bt = lax.axis_index("s") * NUM_CORES + lax.axis_index("c")
        gs, obs, sgs, sos = (g0, g1), (ob0, ob1), (sg0, sg1), (so0, so1)

        def fire_gather(s, b):
            pltpu.async_copy(table_hbm.at[idxall.at[s]], gs[b], sgs[b])

        def wait_gather(b):
            pltpu.make_async_copy(table_hbm.at[idxall.at[0]], gs[b],
                                  sgs[b]).wait()

        def fire_out(s, b):
            for ad in range(8):
                pltpu.async_copy(obs[b].at[pl.ds(8 * ad, 8)],
                                 out_hbm.at[s, ad, bt], sos[b])

        def wait_out(b):
            for ad in range(8):
                pltpu.make_async_copy(obs[b].at[pl.ds(0, 8)],
                                      out_hbm.at[0, 0, bt], sos[b]).wait()

        pltpu.sync_copy(tokp_hbm.at[:, pl.ds(bt * LANE, LANE)], idxall)
        pltpu.sync_copy(tokh_hbm.at[:, pl.ds(bt * LANE, LANE)], hall)

        iota = lax.iota(jnp.int32, 16)
        rows_vs = [iota + 16 * rb for rb in range(8)]
        perms = [(iota + d) % 16 for d in range(16)]

        fire_gather(0, 0)

        @pl.loop(0, seq, step=2)
        def _(s0):
            for b in (0, 1):
                s = s0 + b
                nb = 1 - b
                wait_gather(b)

                @pl.when(s + 1 < seq)
                def _():
                    fire_gather(s + 1, nb)

                @pl.when(s >= 2)
                def _():
                    wait_out(b)

                @plsc.parallel_loop(0, 512, unroll=8)
                def _(i):
                    rb16 = (i >> 2) & 0x70
                    rows_v = iota + rb16
                    cols_v = ((iota + (i & 15)) & 15) | (i & 0x30)
                    hv = hall[s, pl.ds(rb16, 16)]
                    v = plsc.load_gather(gs[b], [rows_v, cols_v + hv])
                    plsc.store_scatter(obs[b], [cols_v, rows_v], v)
                fire_out(s, b)

        wait_out(0)
        wait_out(1)

    return gather_kernel


def kernel(token_ids, embedding_weight):
    batch, seq = token_ids.shape
    vocab, d = embedding_weight.shape
    tokt = token_ids.T.astype(jnp.int32)
    tokp = tokt >> 1
    tokh = (tokt & 1) << 6
    table2 = embedding_weight.reshape(vocab // 2, 2 * d)
    out5 = _build(seq, batch, vocab)(tokp, tokh, table2)
    return out5.transpose(2, 4, 0, 1, 3).reshape(batch, seq, d)


# speedup vs baseline: 1.3943x; 1.3943x over previous
import functools

import jax
import jax.numpy as jnp
from jax import lax
from jax.experimental import pallas as pl
from jax.experimental.pallas import tpu as pltpu
from jax.experimental.pallas import tpu_sc as plsc

D_MODEL = 64
LANE = 128
NUM_CORES = 2
NUM_SUBCORES = 16
NUM_WORKERS = NUM_CORES * NUM_SUBCORES


@functools.cache
def _build_trans(vocab):
    full_tiles = vocab // LANE - (1 if vocab % LANE else 0)
    tail_rows = vocab - full_tiles * LANE

    mesh = plsc.VectorSubcoreMesh(core_axis_name="c", subcore_axis_name="s")

    @functools.partial(
        pl.kernel,
        mesh=mesh,
        compiler_params=pltpu.CompilerParams(needs_layout_passes=False),
        out_type=jax.ShapeDtypeStruct((vocab, LANE), jnp.float32),
        scratch_types=[
            pltpu.VMEM((D_MODEL, LANE), jnp.float32),
            pltpu.VMEM((D_MODEL, LANE), jnp.float32),
            pltpu.VMEM((LANE, LANE), jnp.float32),
            pltpu.VMEM((LANE, LANE), jnp.float32),
            pltpu.SemaphoreType.DMA,
            pltpu.SemaphoreType.DMA,
            pltpu.SemaphoreType.DMA,
            pltpu.SemaphoreType.DMA,
        ],
    )
    def trans_kernel(tabt_hbm, tail_hbm, out_hbm,
                     gi0, gi1, go0, go1, si0, si1, so0, so1):
        w = lax.axis_index("s") * NUM_CORES + lax.axis_index("c")
        gis, gos = (gi0, gi1), (go0, go1)
        sis, sos = (si0, si1), (so0, so1)
        my_jobs = (full_tiles - w + NUM_WORKERS - 1) // NUM_WORKERS

        iota = lax.iota(jnp.int32, 16)

        def fire_in(k, b):
            t = w + k * NUM_WORKERS
            for a in range(8):
                pltpu.async_copy(
                    tabt_hbm.at[pl.ds(8 * a, 8), pl.ds(t * LANE, LANE)],
                    gis[b].at[pl.ds(8 * a, 8)], sis[b])

        def wait_in(b):
            for a in range(8):
                pltpu.make_async_copy(
                    tabt_hbm.at[pl.ds(8 * a, 8), pl.ds(0, LANE)],
                    gis[b].at[pl.ds(8 * a, 8)], sis[b]).wait()

        def fire_out(k, b):
            t = w + k * NUM_WORKERS
            pltpu.async_copy(gos[b], out_hbm.at[pl.ds(t * LANE, LANE)],
                             sos[b])

        def wait_out(b):
            pltpu.make_async_copy(gos[b], out_hbm.at[pl.ds(0, LANE)],
                                  sos[b]).wait()

        @pl.when(w == 0)
        def _():
            pltpu.sync_copy(tail_hbm,
                            out_hbm.at[pl.ds(full_tiles * LANE, tail_rows)])

        fire_in(0, 0)

        n_iters = (full_tiles // NUM_WORKERS + 2) // 2 * 2

        @pl.loop(0, n_iters, step=2)
        def _(k0):
            for b in (0, 1):
                k = k0 + b
                nb = 1 - b

                @pl.when(k < my_jobs)
                def _():
                    wait_in(b)

                    @pl.when(k + 1 < my_jobs)
                    def _():
                        fire_in(k + 1, nb)

                    @pl.when(k >= 2)
                    def _():
                        wait_out(b)

                    @plsc.parallel_loop(0, 512, unroll=8)
                    def _(i):
                        viota = iota + ((i >> 2) & 0x70)
                        dperm = ((iota + (i & 15)) & 15) | (i & 0x30)
                        v = plsc.load_gather(gis[b], [dperm, viota])
                        plsc.store_scatter(gos[b], [viota, dperm], v)
                    fire_out(k, b)

        for b in (0, 1):
            pending = (((my_jobs >= 1) & (((my_jobs - 1) % 2) == b))
                       | ((my_jobs >= 2) & ((my_jobs % 2) == b)))

            @pl.when(pending)
            def _(b=b):
                wait_out(b)

    return trans_kernel


@functools.cache
def _build(seq, batch, vocab):
    assert batch == LANE * NUM_WORKERS and seq % 2 == 0
    n_bt = batch // LANE

    mesh = plsc.VectorSubcoreMesh(core_axis_name="c", subcore_axis_name="s")

    @functools.partial(
        pl.kernel,
        mesh=mesh,
        compiler_params=pltpu.CompilerParams(needs_layout_passes=False),
        out_type=jax.ShapeDtypeStruct((seq, 8, n_bt, 8, LANE), jnp.float32),
        scratch_types=[
            pltpu.VMEM((seq, LANE), jnp.int32),
            pltpu.VMEM((LANE, LANE), jnp.float32),
            pltpu.VMEM((LANE, LANE), jnp.float32),
            pltpu.VMEM((D_MODEL, LANE), jnp.float32),
            pltpu.VMEM((D_MODEL, LANE), jnp.float32),
            pltpu.SemaphoreType.DMA,
            pltpu.SemaphoreType.DMA,
            pltpu.SemaphoreType.DMA,
            pltpu.SemaphoreType.DMA,
        ],
    )
    def gather_kernel(tokt_hbm, table_hbm, out_hbm,
                      idxall, g0, g1, ob0, ob1, sg0, sg1, so0, so1):
        bt = lax.axis_index("s") * NUM_CORES + lax.axis_index("c")
        gs, obs, sgs, sos = (g0, g1), (ob0, ob1), (sg0, sg1), (so0, so1)

        def fire_gather(s, b):
            pltpu.async_copy(table_hbm.at[idxall.at[s]], gs[b], sgs[b])

        def wait_gather(b):
            pltpu.make_async_copy(table_hbm.at[idxall.at[0]], gs[b],
                                  sgs[b]).wait()

        def fire_out(s, b):
            for ad in range(8):
                pltpu.async_copy(obs[b].at[pl.ds(8 * ad, 8)],
                                 out_hbm.at[s, ad, bt], sos[b])

        def wait_out(b):
            for ad in range(8):
                pltpu.make_async_copy(obs[b].at[pl.ds(0, 8)],
                                      out_hbm.at[0, 0, bt], sos[b]).wait()

        pltpu.sync_copy(tokt_hbm.at[:, pl.ds(bt * LANE, LANE)], idxall)

        iota = lax.iota(jnp.int32, 16)
        rows_vs = [iota + 16 * rb for rb in range(8)]
        perms = [(iota + d) % 16 for d in range(16)]

        fire_gather(0, 0)

        @pl.loop(0, seq, step=2)
        def _(s0):
            for b in (0, 1):
                s = s0 + b
                nb = 1 - b
                wait_gather(b)

                @pl.when(s + 1 < seq)
                def _():
                    fire_gather(s + 1, nb)

                @pl.when(s >= 2)
                def _():
                    wait_out(b)

                @plsc.parallel_loop(0, 512, unroll=8)
                def _(i):
                    rows_v = iota + ((i >> 2) & 0x70)
                    cols_v = ((iota + (i & 15)) & 15) | (i & 0x30)
                    v = plsc.load_gather(gs[b], [rows_v, cols_v])
                    plsc.store_scatter(obs[b], [cols_v, rows_v], v)
                fire_out(s, b)

        wait_out(0)
        wait_out(1)

    return gather_kernel


def kernel(token_ids, embedding_weight):
    batch, seq = token_ids.shape
    vocab, d = embedding_weight.shape
    tokt = token_ids.T.astype(jnp.int32)
    full = (vocab // LANE - (1 if vocab % LANE else 0)) * LANE
    tail128 = jnp.pad(embedding_weight[full:], ((0, 0), (0, LANE - d)))
    table128 = _build_trans(vocab)(embedding_weight.T, tail128)
    out5 = _build(seq, batch, vocab)(tokt, table128)
    return out5.transpose(2, 4, 0, 1, 3).reshape(batch, seq, d)
